# baseline (speedup 1.0000x reference)
import functools

import jax
import jax.numpy as jnp
from jax import lax
from jax.experimental import pallas as pl
from jax.experimental.pallas import tpu as pltpu
from jax.experimental.pallas import tpu_sc as plsc

KNN = 3
EPS = 1e-8
L = 16
NC = 2
NS = 16
NW = NC * NS


@functools.partial(jax.jit, static_argnums=(0, 1, 2, 3))
def _sc_call(S, Q, C, NN, s_feats, q_points, s_points, neighbor_indices):
    QPW = Q // NW
    FB = 32
    NFB = QPW // FB
    CBN = C // L

    mesh = plsc.VectorSubcoreMesh(core_axis_name="c", subcore_axis_name="s")

    @functools.partial(
        pl.kernel,
        out_type=jax.ShapeDtypeStruct((Q, C), jnp.float32),
        mesh=mesh,
        compiler_params=pltpu.CompilerParams(needs_layout_passes=False),
        scratch_types=[
            pltpu.VMEM((QPW,), jnp.int32),
            pltpu.VMEM((QPW,), jnp.int32),
            pltpu.VMEM((QPW,), jnp.int32),
            pltpu.VMEM((KNN, QPW), jnp.float32),
            pltpu.SemaphoreType.DMA,
            pltpu.SemaphoreType.DMA,
            pltpu.SemaphoreType.DMA,
            pltpu.SemaphoreType.DMA,
        ],
    )
    def knn_kernel(feats_hbm, qp_hbm, sp_hbm, ni_hbm, out_hbm,
                   idx0_v, idx1_v, idx2_v, w_v, sgA, sgB, soA, soB):
        wid = lax.axis_index("s") * NC + lax.axis_index("c")
        base = wid * QPW
        idx_refs = (idx0_v, idx1_v, idx2_v)

        lanes = lax.iota(jnp.int32, L)
        zero_i = jnp.zeros((L,), jnp.int32)

        def _phase1(ni_v, qp_v, sp_v):
            pltpu.sync_copy(ni_hbm.at[pl.ds(base * NN, QPW * NN)], ni_v)
            pltpu.sync_copy(qp_hbm.at[pl.ds(base * 3, QPW * 3)], qp_v)
            pltpu.sync_copy(sp_hbm, sp_v)

            @pl.loop(0, QPW // L)
            def _p1(g):
                sl = pl.ds(g * L, L)
                jv = lanes + g * L
                jv3 = jv * 3
                jvn = jv * NN
                qxv = plsc.load_gather(qp_v, [jv3])
                qyv = plsc.load_gather(qp_v, [jv3 + 1])
                qzv = plsc.load_gather(qp_v, [jv3 + 2])
                ws = []
                for k in range(KNN):
                    iv = plsc.load_gather(ni_v, [jvn + k])
                    idx_refs[k][sl] = iv
                    iv3 = iv * 3
                    sx = plsc.load_gather(sp_v, [iv3])
                    sy = plsc.load_gather(sp_v, [iv3 + 1])
                    sz = plsc.load_gather(sp_v, [iv3 + 2])
                    dx = qxv - sx
                    dy = qyv - sy
                    dz = qzv - sz
                    d2 = dx * dx + dy * dy + dz * dz
                    ws.append(1.0 / (d2 + EPS))
                wsum = ws[0] + ws[1] + ws[2]
                for k in range(KNN):
                    w_v[k, sl] = ws[k] / wsum

        pl.run_scoped(
            _phase1,
            pltpu.VMEM((QPW * NN,), jnp.int32),
            pltpu.VMEM((QPW * 3,), jnp.float32),
            pltpu.VMEM((S * 3,), jnp.float32),
        )

        def _issue(qb, r, sg):
            for k in range(KNN):
                pltpu.async_copy(
                    feats_hbm.at[idx_refs[k].at[pl.ds(qb, FB)]], r.at[k], sg)

        def _wait_g(qb, r, sg):
            for k in range(KNN):
                pltpu.make_async_copy(
                    feats_hbm.at[idx_refs[k].at[pl.ds(qb, FB)]], r.at[k],
                    sg).wait()

        def _wait_o(o, so):
            pltpu.make_async_copy(o, out_hbm.at[pl.ds(base, FB)], so).wait()

        def _compute(qb, r, o):
            @pl.loop(0, FB)
            def _q(qi):
                widx = jnp.full((L,), qb + qi, dtype=jnp.int32)
                w0 = plsc.load_gather(w_v, [zero_i, widx])
                w1 = plsc.load_gather(w_v, [zero_i + 1, widx])
                w2 = plsc.load_gather(w_v, [zero_i + 2, widx])
                for cb in range(CBN):
                    cs = pl.ds(cb * L, L)
                    o[qi, cs] = (w0 * r[0, qi, cs]
                                 + w1 * r[1, qi, cs]
                                 + w2 * r[2, qi, cs])

        def _phase2(rA, rB, outA, outB):
            slots = ((rA, outA, sgA, soA), (rB, outB, sgB, soB))
            _issue(0, rA, sgA)
            _issue(FB, rB, sgB)

            @pl.loop(0, NFB // 2)
            def _p2(p):
                for off, (r, o, sg, so) in enumerate(slots):
                    qb = (2 * p + off) * FB
                    _wait_g(qb, r, sg)

                    @pl.when(p > 0)
                    def _():
                        _wait_o(o, so)

                    _compute(qb, r, o)
                    pltpu.async_copy(o, out_hbm.at[pl.ds(base + qb, FB)], so)
                    nqb = qb + 2 * FB

                    @pl.when(nqb < QPW)
                    def _():
                        _issue(nqb, r, sg)

            _wait_o(outA, soA)
            _wait_o(outB, soB)

        pl.run_scoped(
            _phase2,
            pltpu.VMEM((KNN, FB, C), jnp.float32),
            pltpu.VMEM((KNN, FB, C), jnp.float32),
            pltpu.VMEM((FB, C), jnp.float32),
            pltpu.VMEM((FB, C), jnp.float32),
        )

    return knn_kernel(s_feats, q_points.reshape(-1), s_points.reshape(-1),
                      neighbor_indices.reshape(-1))


def kernel(s_feats, q_points, s_points, neighbor_indices):
    S, C = s_feats.shape
    Q, NN = neighbor_indices.shape
    return _sc_call(S, Q, C, NN, s_feats.astype(jnp.float32),
                    q_points.astype(jnp.float32),
                    s_points.astype(jnp.float32),
                    neighbor_indices.astype(jnp.int32))


# speedup vs baseline: 1.6623x; 1.6623x over previous
import functools

import jax
import jax.numpy as jnp
from jax import lax
from jax.experimental import pallas as pl
from jax.experimental.pallas import tpu as pltpu
from jax.experimental.pallas import tpu_sc as plsc

KNN = 3
EPS = 1e-8
L = 16
NC = 2
NS = 16
NW = NC * NS
NSLOT = 4


@functools.partial(jax.jit, static_argnums=(0, 1, 2))
def _sc_call(S, Q, C, s_feats, qx, qy, qz, spx, spy, spz, i0, i1, i2):
    QPW = Q // NW
    FB = 32
    NFB = QPW // FB
    CBN = C // L

    mesh = plsc.VectorSubcoreMesh(core_axis_name="c", subcore_axis_name="s")

    @functools.partial(
        pl.kernel,
        out_type=jax.ShapeDtypeStruct((Q, C), jnp.float32),
        mesh=mesh,
        compiler_params=pltpu.CompilerParams(needs_layout_passes=False),
        scratch_types=[
            pltpu.VMEM((QPW,), jnp.int32),
            pltpu.VMEM((QPW,), jnp.int32),
            pltpu.VMEM((QPW,), jnp.int32),
            pltpu.VMEM((KNN, QPW), jnp.float32),
            [pltpu.SemaphoreType.DMA] * NSLOT,
            [pltpu.SemaphoreType.DMA] * 2,
        ],
    )
    def knn_kernel(feats_hbm, qx_hbm, qy_hbm, qz_hbm, spx_hbm, spy_hbm,
                   spz_hbm, i0_hbm, i1_hbm, i2_hbm, out_hbm,
                   idx0_v, idx1_v, idx2_v, w_v, sgs, sos):
        wid = lax.axis_index("s") * NC + lax.axis_index("c")
        base = wid * QPW
        idx_refs = (idx0_v, idx1_v, idx2_v)

        for k, src in enumerate((i0_hbm, i1_hbm, i2_hbm)):
            pltpu.sync_copy(src.at[pl.ds(base, QPW)], idx_refs[k])

        zero_i = jnp.zeros((L,), jnp.int32)

        def _phase1(qx_v, qy_v, qz_v, spx_v, spy_v, spz_v):
            pltpu.sync_copy(qx_hbm.at[pl.ds(base, QPW)], qx_v)
            pltpu.sync_copy(qy_hbm.at[pl.ds(base, QPW)], qy_v)
            pltpu.sync_copy(qz_hbm.at[pl.ds(base, QPW)], qz_v)
            pltpu.sync_copy(spx_hbm, spx_v)
            pltpu.sync_copy(spy_hbm, spy_v)
            pltpu.sync_copy(spz_hbm, spz_v)

            @pl.loop(0, QPW // L)
            def _p1(g):
                sl = pl.ds(g * L, L)
                qxv = qx_v[sl]
                qyv = qy_v[sl]
                qzv = qz_v[sl]
                ws = []
                for k in range(KNN):
                    iv = idx_refs[k][sl]
                    sx = plsc.load_gather(spx_v, [iv])
                    sy = plsc.load_gather(spy_v, [iv])
                    sz = plsc.load_gather(spz_v, [iv])
                    dx = qxv - sx
                    dy = qyv - sy
                    dz = qzv - sz
                    d2 = dx * dx + dy * dy + dz * dz
                    ws.append(1.0 / (d2 + EPS))
                wsum = ws[0] + ws[1] + ws[2]
                for k in range(KNN):
                    w_v[k, sl] = ws[k] / wsum

        pl.run_scoped(
            _phase1,
            pltpu.VMEM((QPW,), jnp.float32),
            pltpu.VMEM((QPW,), jnp.float32),
            pltpu.VMEM((QPW,), jnp.float32),
            pltpu.VMEM((S,), jnp.float32),
            pltpu.VMEM((S,), jnp.float32),
            pltpu.VMEM((S,), jnp.float32),
        )

        def _issue(qb, r, sg):
            for k in range(KNN):
                pltpu.async_copy(
                    feats_hbm.at[idx_refs[k].at[pl.ds(qb, FB)]], r.at[k], sg)

        def _wait_g(qb, r, sg):
            for k in range(KNN):
                pltpu.make_async_copy(
                    feats_hbm.at[idx_refs[k].at[pl.ds(qb, FB)]], r.at[k],
                    sg).wait()

        def _wait_o(o, so):
            pltpu.make_async_copy(o, out_hbm.at[pl.ds(base, FB)], so).wait()

        def _compute(qb, r, o):
            @pl.loop(0, FB)
            def _q(qi):
                widx = jnp.full((L,), qb + qi, dtype=jnp.int32)
                w0 = plsc.load_gather(w_v, [zero_i, widx])
                w1 = plsc.load_gather(w_v, [zero_i + 1, widx])
                w2 = plsc.load_gather(w_v, [zero_i + 2, widx])
                for cb in range(CBN):
                    cs = pl.ds(cb * L, L)
                    o[qi, cs] = (w0 * r[0, qi, cs]
                                 + w1 * r[1, qi, cs]
                                 + w2 * r[2, qi, cs])

        def _phase2(rs, outs):
            for s in range(NSLOT):
                _issue(s * FB, rs[s], sgs[s])

            @pl.loop(0, NFB // NSLOT)
            def _p2(p):
                for s in range(NSLOT):
                    o, so = outs[s % 2], sos[s % 2]
                    qb = (NSLOT * p + s) * FB
                    _wait_g(qb, rs[s], sgs[s])
                    if s >= 2:
                        _wait_o(o, so)
                    else:
                        @pl.when(p > 0)
                        def _():
                            _wait_o(o, so)

                    _compute(qb, rs[s], o)
                    pltpu.async_copy(o, out_hbm.at[pl.ds(base + qb, FB)], so)
                    nqb = qb + NSLOT * FB

                    @pl.when(nqb < QPW)
                    def _():
                        _issue(nqb, rs[s], sgs[s])

            _wait_o(outs[0], sos[0])
            _wait_o(outs[1], sos[1])

        pl.run_scoped(
            _phase2,
            [pltpu.VMEM((KNN, FB, C), jnp.float32)] * NSLOT,
            [pltpu.VMEM((FB, C), jnp.float32)] * 2,
        )

    return knn_kernel(s_feats, qx, qy, qz, spx, spy, spz, i0, i1, i2)


def kernel(s_feats, q_points, s_points, neighbor_indices):
    S, C = s_feats.shape
    Q = q_points.shape[0]
    qp = q_points.astype(jnp.float32)
    sp = s_points.astype(jnp.float32)
    ni = neighbor_indices.astype(jnp.int32)
    return _sc_call(S, Q, C, s_feats.astype(jnp.float32),
                    qp[:, 0], qp[:, 1], qp[:, 2],
                    sp[:, 0], sp[:, 1], sp[:, 2],
                    ni[:, 0], ni[:, 1], ni[:, 2])
